# baseline (speedup 1.0000x reference)
import jax
import jax.numpy as jnp
from jax.experimental import pallas as pl
from jax.experimental.pallas import tpu as pltpu

_BI = 200


def _gcn_block(seq_ref, w_ref, adj_ref, out_ref, support_ref):
    @pl.when(pl.program_id(0) == 0)
    def _():
        support_ref[...] = jnp.dot(
            seq_ref[...], w_ref[...], preferred_element_type=jnp.float32
        )

    out_ref[...] = jnp.tanh(
        jnp.dot(adj_ref[...], support_ref[...], preferred_element_type=jnp.float32)
    )


def kernel(seq, adj, weight):
    n, in_ft = seq.shape
    out_ft = weight.shape[1]
    return pl.pallas_call(
        _gcn_block,
        grid=(n // _BI,),
        in_specs=[
            pl.BlockSpec((n, in_ft), lambda i: (0, 0)),
            pl.BlockSpec((in_ft, out_ft), lambda i: (0, 0)),
            pl.BlockSpec((_BI, n), lambda i: (i, 0)),
        ],
        out_specs=pl.BlockSpec((_BI, out_ft), lambda i: (i, 0)),
        out_shape=jax.ShapeDtypeStruct((n, out_ft), jnp.float32),
        scratch_shapes=[pltpu.VMEM((n, out_ft), jnp.float32)],
    )(seq, weight, adj)


# speedup vs baseline: 1.0174x; 1.0174x over previous
import jax
import jax.numpy as jnp
from jax.experimental import pallas as pl

_BI = 400


def _gcn_block(seq_ref, w_ref, adj_ref, out_ref):
    tmp = jnp.dot(adj_ref[...], seq_ref[...], preferred_element_type=jnp.float32)
    out_ref[...] = jnp.tanh(
        jnp.dot(tmp, w_ref[...], preferred_element_type=jnp.float32)
    )


def kernel(seq, adj, weight):
    n, in_ft = seq.shape
    out_ft = weight.shape[1]
    return pl.pallas_call(
        _gcn_block,
        grid=(n // _BI,),
        in_specs=[
            pl.BlockSpec((n, in_ft), lambda i: (0, 0)),
            pl.BlockSpec((in_ft, out_ft), lambda i: (0, 0)),
            pl.BlockSpec((_BI, n), lambda i: (i, 0)),
        ],
        out_specs=pl.BlockSpec((_BI, out_ft), lambda i: (i, 0)),
        out_shape=jax.ShapeDtypeStruct((n, out_ft), jnp.float32),
    )(seq, weight, adj)
